# Initial kernel scaffold; baseline (speedup 1.0000x reference)
import jax
import jax.numpy as jnp
from jax.experimental import pallas as pl


def kernel(z_e, phi, W1, b1, W2, b2, W3, b3, codebooks):
    raise NotImplementedError("write your pallas kernel here")


# speedup vs baseline: 4.7508x; 4.7508x over previous
import math

import jax
import jax.numpy as jnp
from jax.experimental import pallas as pl
from jax.experimental.pallas import tpu as pltpu

_B, _N, _C = 4, 4096, 32
_DPHI, _H1, _H2 = 2048, 128, 128
_R, _K, _MPHY = 8, 1024, 4
_J = _R * _MPHY
_BETA = 0.25
_BN = 1024
_NBLK = _N // _BN
_QINV = 1.0 / math.sqrt(10.0 + 1e-9)
_LOSS_SCALE = (1.0 + _BETA) / float(_B * _N * _C * _C)


def _router_body(phi_ref, w1_ref, b1_ref, w2_ref, b2_ref, w3_ref, b3_ref,
                 logits_ref, probs_ref, modes_ref):
    f32 = jnp.float32
    dn = (((1,), (0,)), ((), ()))
    h = jnp.maximum(
        jax.lax.dot_general(phi_ref[...], w1_ref[...], dn,
                            preferred_element_type=f32) + b1_ref[...], 0.0)
    h = jnp.maximum(
        jax.lax.dot_general(h, w2_ref[...], dn,
                            preferred_element_type=f32) + b2_ref[...], 0.0)
    logits = jax.lax.dot_general(h, w3_ref[...], dn,
                                 preferred_element_type=f32) + b3_ref[...]
    logits_ref[...] = logits
    mx = jnp.max(logits, axis=-1, keepdims=True)
    ex = jnp.exp(logits - mx)
    probs_ref[...] = ex / jnp.sum(ex, axis=-1, keepdims=True)
    lane = jax.lax.broadcasted_iota(jnp.int32, (_B, _J), 1)
    mode = jnp.min(jnp.where(logits == mx, lane, _J), axis=-1, keepdims=True)
    expert = mode // _MPHY
    modes_ref[:, 0:1] = mode
    modes_ref[:, 1:2] = expert
    modes_ref[:, 2:3] = mode - _MPHY * expert
    modes_ref[:, 3:4] = mode


def _vq_body(expert_ref, z_ref, cb_ref,
             idx_ref, gidx_ref, zq_ref, loss_ref):
    b = pl.program_id(0)
    j = pl.program_id(1)
    zb = z_ref[0]
    cb = cb_ref[0]
    crossT = jax.lax.dot_general(cb, zb, (((1,), (1,)), ((), ())),
                                 preferred_element_type=jnp.float32)
    e_sq = jnp.sum(cb * cb, axis=-1, keepdims=True)
    z_sq_row = jnp.sum(zb * zb, axis=-1, keepdims=True).T
    dT = z_sq_row + e_sq - 2.0 * crossT
    dmin = jnp.min(dT, axis=0, keepdims=True)
    kiota = jax.lax.broadcasted_iota(jnp.int32, (_K, _BN), 0)
    sel = jnp.where(dT == dmin, kiota, _K)
    idx_row = jnp.min(sel, axis=0, keepdims=True)
    onehotT = (kiota == idx_row).astype(jnp.float32)
    zq_blk = jax.lax.dot_general(onehotT, cb, (((0,), (0,)), ((), ())),
                                 preferred_element_type=jnp.float32)
    diff = zq_blk - zb
    part = jnp.sum(diff * diff)

    @pl.when(jnp.logical_and(b == 0, j == 0))
    def _():
        loss_ref[...] = jnp.zeros((1, 1), jnp.float32)

    loss_ref[...] += part.reshape(1, 1)

    @pl.when(jnp.logical_and(b == _B - 1, j == _NBLK - 1))
    def _():
        loss_ref[...] = loss_ref[...] * _LOSS_SCALE

    idx_ref[pl.ds(b, 1), pl.ds(j * _BN, _BN)] = idx_row
    gidx_ref[pl.ds(b, 1), pl.ds(j * _BN, _BN)] = idx_row + expert_ref[b] * _K
    zq_ref[0] = zb + (zq_blk - zb)


def _sym_body(i0_ref, i1_ref, out_ref):
    i0 = i0_ref[...]
    i1 = i1_ref[...]
    s_list = [
        i0 >> 6,
        (i0 >> 2) & 15,
        ((i0 & 3) << 2) | (i1 >> 8),
        (i1 >> 4) & 15,
        i1 & 15,
    ]
    for t in range(5):
        st = s_list[t]
        out_ref[t, 0] = ((st >> 2) * 2 - 3).astype(jnp.float32) * _QINV
        out_ref[t, 1] = ((st & 3) * 2 - 3).astype(jnp.float32) * _QINV


def kernel(z_e, phi, W1, b1, W2, b2, W3, b3, codebooks):
    f32 = jnp.float32
    logits, probs, modes = pl.pallas_call(
        _router_body,
        out_shape=[
            jax.ShapeDtypeStruct((_B, _J), f32),
            jax.ShapeDtypeStruct((_B, _J), f32),
            jax.ShapeDtypeStruct((_B, 4), jnp.int32),
        ],
    )(phi, W1, b1.reshape(1, _H1), W2, b2.reshape(1, _H2), W3,
      b3.reshape(1, _J))
    mode_idx = modes[:, 0]
    expert_idx = modes[:, 1]
    phy_idx = modes[:, 2]

    grid_spec = pltpu.PrefetchScalarGridSpec(
        num_scalar_prefetch=1,
        grid=(_B, _NBLK),
        in_specs=[
            pl.BlockSpec((1, _BN, _C), lambda b, j, e: (b, j, 0)),
            pl.BlockSpec((1, _K, _C), lambda b, j, e: (e[b], 0, 0)),
        ],
        out_specs=[
            pl.BlockSpec((_B, _N), lambda b, j, e: (0, 0)),
            pl.BlockSpec((_B, _N), lambda b, j, e: (0, 0)),
            pl.BlockSpec((1, _BN, _C), lambda b, j, e: (b, j, 0)),
            pl.BlockSpec((1, 1), lambda b, j, e: (0, 0)),
        ],
    )
    indices, gidx, z_q_st, loss = pl.pallas_call(
        _vq_body,
        grid_spec=grid_spec,
        out_shape=[
            jax.ShapeDtypeStruct((_B, _N), jnp.int32),
            jax.ShapeDtypeStruct((_B, _N), jnp.int32),
            jax.ShapeDtypeStruct((_B, _N, _C), f32),
            jax.ShapeDtypeStruct((1, 1), f32),
        ],
    )(expert_idx, z_e, codebooks)
    vq_loss = loss[0, 0]

    pairs = indices.reshape(_B, _N // 2, 2)
    symout = pl.pallas_call(
        _sym_body,
        out_shape=jax.ShapeDtypeStruct((5, 2, _B, _N // 2), f32),
    )(pairs[:, :, 0], pairs[:, :, 1])
    symbols = symout.transpose(2, 3, 0, 1).reshape(_B, _N * 10 // 4, 2)

    return (z_q_st, indices, vq_loss, logits, probs, mode_idx, phy_idx,
            symbols)


# speedup vs baseline: 5.3178x; 1.1194x over previous
import math

import jax
import jax.numpy as jnp
from jax.experimental import pallas as pl
from jax.experimental.pallas import tpu as pltpu

_B, _N, _C = 4, 4096, 32
_DPHI, _H1, _H2 = 2048, 128, 128
_R, _K, _MPHY = 8, 1024, 4
_J = _R * _MPHY
_BETA = 0.25
_BN = 1024
_NBLK = _N // _BN
_QINV = 1.0 / math.sqrt(10.0 + 1e-9)
_LOSS_SCALE = (1.0 + _BETA) / float(_B * _N * _C * _C)


def _router_body(phi_ref, w1_ref, b1_ref, w2_ref, b2_ref, w3_ref, b3_ref,
                 logits_ref, probs_ref, modes_ref):
    f32 = jnp.float32
    dn = (((1,), (0,)), ((), ()))
    h = jnp.maximum(
        jax.lax.dot_general(phi_ref[...], w1_ref[...], dn,
                            preferred_element_type=f32) + b1_ref[...], 0.0)
    h = jnp.maximum(
        jax.lax.dot_general(h, w2_ref[...], dn,
                            preferred_element_type=f32) + b2_ref[...], 0.0)
    logits = jax.lax.dot_general(h, w3_ref[...], dn,
                                 preferred_element_type=f32) + b3_ref[...]
    logits_ref[...] = logits
    mx = jnp.max(logits, axis=-1, keepdims=True)
    ex = jnp.exp(logits - mx)
    probs_ref[...] = ex / jnp.sum(ex, axis=-1, keepdims=True)
    lane = jax.lax.broadcasted_iota(jnp.int32, (_B, _J), 1)
    mode = jnp.min(jnp.where(logits == mx, lane, _J), axis=-1, keepdims=True)
    expert = mode // _MPHY
    modes_ref[:, 0:1] = mode
    modes_ref[:, 1:2] = expert
    modes_ref[:, 2:3] = mode - _MPHY * expert
    modes_ref[:, 3:4] = mode


def _vq_body(expert_ref, z_ref, cb_ref,
             idx_ref, gidx_ref, zq_ref, loss_ref):
    b = pl.program_id(0)
    j = pl.program_id(1)
    zb = z_ref[0]
    cb = cb_ref[0]
    crossT = jax.lax.dot_general(cb, zb, (((1,), (1,)), ((), ())),
                                 preferred_element_type=jnp.float32)
    e_sq = jnp.sum(cb * cb, axis=-1, keepdims=True)
    z_sq_row = jnp.sum(zb * zb, axis=-1, keepdims=True).T
    dT = z_sq_row + e_sq - 2.0 * crossT
    idx_row = jnp.argmin(dT, axis=0)[None, :]
    kiota = jax.lax.broadcasted_iota(jnp.int32, (_K, _BN), 0)
    onehotT = (kiota == idx_row).astype(jnp.float32)
    zq_blk = jax.lax.dot_general(onehotT, cb, (((0,), (0,)), ((), ())),
                                 preferred_element_type=jnp.float32)
    diff = zq_blk - zb
    part = jnp.sum(diff * diff)

    @pl.when(jnp.logical_and(b == 0, j == 0))
    def _():
        loss_ref[...] = jnp.zeros((1, 1), jnp.float32)

    loss_ref[...] += part.reshape(1, 1)

    @pl.when(jnp.logical_and(b == _B - 1, j == _NBLK - 1))
    def _():
        loss_ref[...] = loss_ref[...] * _LOSS_SCALE

    idx_ref[pl.ds(b, 1), pl.ds(j * _BN, _BN)] = idx_row
    gidx_ref[pl.ds(b, 1), pl.ds(j * _BN, _BN)] = idx_row + expert_ref[b] * _K
    zq_ref[0] = zb + (zq_blk - zb)


def _sym_body(i0_ref, i1_ref, out_ref):
    i0 = i0_ref[...]
    i1 = i1_ref[...]
    s_list = [
        i0 >> 6,
        (i0 >> 2) & 15,
        ((i0 & 3) << 2) | (i1 >> 8),
        (i1 >> 4) & 15,
        i1 & 15,
    ]
    for t in range(5):
        st = s_list[t]
        out_ref[t, 0] = ((st >> 2) * 2 - 3).astype(jnp.float32) * _QINV
        out_ref[t, 1] = ((st & 3) * 2 - 3).astype(jnp.float32) * _QINV


def kernel(z_e, phi, W1, b1, W2, b2, W3, b3, codebooks):
    f32 = jnp.float32
    logits, probs, modes = pl.pallas_call(
        _router_body,
        out_shape=[
            jax.ShapeDtypeStruct((_B, _J), f32),
            jax.ShapeDtypeStruct((_B, _J), f32),
            jax.ShapeDtypeStruct((_B, 4), jnp.int32),
        ],
    )(phi, W1, b1.reshape(1, _H1), W2, b2.reshape(1, _H2), W3,
      b3.reshape(1, _J))
    mode_idx = modes[:, 0]
    expert_idx = modes[:, 1]
    phy_idx = modes[:, 2]

    grid_spec = pltpu.PrefetchScalarGridSpec(
        num_scalar_prefetch=1,
        grid=(_B, _NBLK),
        in_specs=[
            pl.BlockSpec((1, _BN, _C), lambda b, j, e: (b, j, 0)),
            pl.BlockSpec((1, _K, _C), lambda b, j, e: (e[b], 0, 0)),
        ],
        out_specs=[
            pl.BlockSpec((_B, _N), lambda b, j, e: (0, 0)),
            pl.BlockSpec((_B, _N), lambda b, j, e: (0, 0)),
            pl.BlockSpec((1, _BN, _C), lambda b, j, e: (b, j, 0)),
            pl.BlockSpec((1, 1), lambda b, j, e: (0, 0)),
        ],
    )
    indices, gidx, z_q_st, loss = pl.pallas_call(
        _vq_body,
        grid_spec=grid_spec,
        out_shape=[
            jax.ShapeDtypeStruct((_B, _N), jnp.int32),
            jax.ShapeDtypeStruct((_B, _N), jnp.int32),
            jax.ShapeDtypeStruct((_B, _N, _C), f32),
            jax.ShapeDtypeStruct((1, 1), f32),
        ],
    )(expert_idx, z_e, codebooks)
    vq_loss = loss[0, 0]

    pairs = indices.reshape(_B, _N // 2, 2)
    symout = pl.pallas_call(
        _sym_body,
        out_shape=jax.ShapeDtypeStruct((5, 2, _B, _N // 2), f32),
    )(pairs[:, :, 0], pairs[:, :, 1])
    symbols = symout.transpose(2, 3, 0, 1).reshape(_B, _N * 10 // 4, 2)

    return (z_q_st, indices, vq_loss, logits, probs, mode_idx, phy_idx,
            symbols)
